# baseline (speedup 1.0000x reference)
import functools

import jax
import jax.numpy as jnp
from jax import lax
from jax.experimental import pallas as pl
from jax.experimental.pallas import tpu as pltpu
from jax.experimental.pallas import tpu_sc as plsc

N = 10000
E = 320000
H = 8
F = 16
HF = H * F
NEG = 0.2
NC = 2
NS = 16
NW = NC * NS
L = 16
EC = E // NW
BA = 100
NCA = EC // BA
PAIRS_A = BA // 2
BC = 50
NCC = EC // BC
PAIRS_C = BC // 2
NPAD = 10240
ROWS_D = NPAD // NS
ROWS_R = N // NS
DW = NPAD * H // NW

f32 = jnp.float32
i32 = jnp.int32

_mesh = plsc.VectorSubcoreMesh(core_axis_name="c", subcore_axis_name="s")
_CP = pltpu.CompilerParams(needs_layout_passes=False, use_tc_tiling_on_sc=False)


def _tc_prep(feat, W, Alr):
    def body(f_ref, w_ref, alr_ref, ft_ref, elr_ref):
        ftb = jnp.dot(f_ref[...], w_ref[...], preferred_element_type=f32)
        ft_ref[...] = ftb
        elr_ref[...] = jnp.dot(ftb, alr_ref[...], preferred_element_type=f32)

    R = 1000
    return pl.pallas_call(
        body,
        grid=(N // R,),
        in_specs=[
            pl.BlockSpec((R, HF), lambda i: (i, 0)),
            pl.BlockSpec((HF, HF), lambda i: (0, 0)),
            pl.BlockSpec((HF, 2 * H), lambda i: (0, 0)),
        ],
        out_specs=[
            pl.BlockSpec((R, HF), lambda i: (i, 0)),
            pl.BlockSpec((R, 2 * H), lambda i: (i, 0)),
        ],
        out_shape=[
            jax.ShapeDtypeStruct((N, HF), f32),
            jax.ShapeDtypeStruct((N, 2 * H), f32),
        ],
    )(feat, W, Alr)


def _sc_edge1(src3, dst3, elr):
    @functools.partial(
        pl.kernel,
        out_type=(
            jax.ShapeDtypeStruct((E, H), f32),
            jax.ShapeDtypeStruct((NC, NPAD, H), f32),
        ),
        mesh=_mesh,
        scratch_types=[
            pltpu.VMEM((NCA, BA), i32),
            pltpu.VMEM((NCA, BA), i32),
            pltpu.VMEM((BA, 2 * H), f32),
            pltpu.VMEM((BA, 2 * H), f32),
            pltpu.VMEM((BA, 2 * H), f32),
            pltpu.VMEM((BA, 2 * H), f32),
            pltpu.VMEM((BA, H), f32),
            pltpu.VMEM((BA, H), f32),
            pltpu.VMEM_SHARED((NPAD, H), f32),
            pltpu.SemaphoreType.DMA,
            pltpu.SemaphoreType.DMA,
            pltpu.SemaphoreType.DMA,
            pltpu.SemaphoreType.DMA,
        ],
        compiler_params=_CP,
    )
    def k(src3_h, dst3_h, elr_h, ee_h, den_h,
          sidx_all, didx_all, srows0, srows1, drows0, drows1, ee0, ee1,
          den_sp, sg0, sg1, sw0, sw1):
        cid = lax.axis_index("c")
        sid = lax.axis_index("s")
        wid = cid * NS + sid
        srows = [srows0, srows1]
        drows = [drows0, drows1]
        eeb = [ee0, ee1]
        sg = [sg0, sg1]
        sw = [sw0, sw1]
        iot = lax.iota(i32, L)
        rowb = jnp.where(iot >= H, 1, 0).astype(i32)
        colb = iot & (H - 1)
        zed = jnp.zeros((L,), f32)

        @plsc.parallel_loop(0, PAIRS_A, unroll=1)
        def zloop(p):
            rows = jnp.full((L,), 2 * p, i32) + rowb
            plsc.store_scatter(ee0, [rows, colb], zed)

        for t in range(ROWS_D // BA):
            pltpu.sync_copy(ee0, den_sp.at[pl.ds(sid * ROWS_D + t * BA, BA)])
        pltpu.sync_copy(ee0.at[pl.ds(0, ROWS_D % BA)],
                        den_sp.at[pl.ds(sid * ROWS_D + (ROWS_D // BA) * BA,
                                        ROWS_D % BA)])
        pltpu.sync_copy(src3_h.at[wid], sidx_all)
        pltpu.sync_copy(dst3_h.at[wid], didx_all)
        plsc.subcore_barrier()

        def fire_in(kk, s):
            c1 = pltpu.async_copy(elr_h.at[sidx_all.at[kk]], srows[s], sg[s])
            c2 = pltpu.async_copy(elr_h.at[didx_all.at[kk]], drows[s], sg[s])
            return (c1, c2)

        def sync_out(kk, s):
            e0 = wid * EC + kk * BA
            pltpu.sync_copy(eeb[s], ee_h.at[pl.ds(e0, BA)])
            pltpu.sync_copy(eeb[s], den_sp.at[didx_all.at[kk]], add=True)

        def compute(s):
            @plsc.parallel_loop(0, PAIRS_A, unroll=2)
            def pair(p):
                rows = jnp.full((L,), 2 * p, i32) + rowb
                elv = plsc.load_gather(srows[s], [rows, colb])
                erv = plsc.load_gather(drows[s], [rows, colb + H])
                e = elv + erv
                e = jnp.where(e > 0, e, NEG * e)
                plsc.store_scatter(eeb[s], [rows, colb], jnp.exp(e))

        def step(half, c):
            k0 = 2 * half
            k1 = k0 + 1
            cps0 = fire_in(k0, 0)
            cps1 = fire_in(k1, 1)
            for cp in cps0:
                cp.wait()
            compute(0)
            sync_out(k0, 0)
            for cp in cps1:
                cp.wait()
            compute(1)
            sync_out(k1, 1)
            return c

        lax.fori_loop(0, NCA // 2, step, 0)
        plsc.subcore_barrier()
        pltpu.sync_copy(den_sp.at[pl.ds(sid * ROWS_D, ROWS_D)],
                        den_h.at[cid, pl.ds(sid * ROWS_D, ROWS_D)])

    return k(src3, dst3, elr)


def _sc_rdenom(dflat):
    @functools.partial(
        pl.kernel,
        out_type=jax.ShapeDtypeStruct((NPAD * H,), f32),
        mesh=_mesh,
        scratch_types=[
            pltpu.VMEM((DW,), f32),
            pltpu.VMEM((DW,), f32),
            pltpu.VMEM((DW,), f32),
        ],
        compiler_params=_CP,
    )
    def k(d_h, rd_h, b0, b1, ob):
        cid = lax.axis_index("c")
        sid = lax.axis_index("s")
        off = (cid * NS + sid) * DW
        pltpu.sync_copy(d_h.at[0, pl.ds(off, DW)], b0)
        pltpu.sync_copy(d_h.at[1, pl.ds(off, DW)], b1)

        def it(j, c):
            v = b0[pl.ds(j * L, L)] + b1[pl.ds(j * L, L)]
            ob[pl.ds(j * L, L)] = 1.0 / v
            return c

        lax.fori_loop(0, DW // L, it, 0)
        pltpu.sync_copy(ob, rd_h.at[pl.ds(off, DW)])

    return k(dflat)


def _sc_edge2(src3, dst3, ee, rd, ft):
    @functools.partial(
        pl.kernel,
        out_type=(
            jax.ShapeDtypeStruct((E, H), f32),
            jax.ShapeDtypeStruct((NC, N, HF), f32),
        ),
        mesh=_mesh,
        scratch_types=[
            pltpu.VMEM((NCC, BC), i32),
            pltpu.VMEM((NCC, BC), i32),
            pltpu.VMEM((BC, H), f32),
            pltpu.VMEM((BC, H), f32),
            pltpu.VMEM((BC, H), f32),
            pltpu.VMEM((BC, H), f32),
            pltpu.VMEM((BC, H), f32),
            pltpu.VMEM((BC, H), f32),
            pltpu.VMEM((BC, HF), f32),
            pltpu.VMEM((BC, HF), f32),
            pltpu.VMEM((BC, HF), f32),
            pltpu.VMEM((BC, HF), f32),
            pltpu.VMEM_SHARED((N, HF), f32),
            pltpu.SemaphoreType.DMA,
            pltpu.SemaphoreType.DMA,
            pltpu.SemaphoreType.DMA,
            pltpu.SemaphoreType.DMA,
        ],
        compiler_params=_CP,
    )
    def k(src3_h, dst3_h, ee_h, rd_h, ft_h, a_h, rst_h,
          sidx_all, didx_all, ee0, ee1, rd0, rd1, rd2, rd3,
          ft0, ft1, ft2, ft3, rst_sp, sg0, sg1, sg2, sg3):
        cid = lax.axis_index("c")
        sid = lax.axis_index("s")
        wid = cid * NS + sid
        zed = jnp.zeros((L,), f32)

        @plsc.parallel_loop(0, BC, unroll=1)
        def zloop(j):
            for jj in range(H):
                ft0[j, pl.ds(jj * F, F)] = zed

        for t in range(ROWS_R // BC):
            pltpu.sync_copy(ft0, rst_sp.at[pl.ds(sid * ROWS_R + t * BC, BC)])
        pltpu.sync_copy(ft0.at[pl.ds(0, ROWS_R % BC)],
                        rst_sp.at[pl.ds(sid * ROWS_R + (ROWS_R // BC) * BC,
                                        ROWS_R % BC)])
        pltpu.sync_copy(src3_h.at[wid], sidx_all)
        pltpu.sync_copy(dst3_h.at[wid], didx_all)
        plsc.subcore_barrier()
        eeb = [ee0, ee1]
        rdb = [rd0, rd1, rd2, rd3]
        ftb = [ft0, ft1, ft2, ft3]
        sg = [sg0, sg1, sg2, sg3]
        iot = lax.iota(i32, L)
        rowb = jnp.where(iot >= H, 1, 0).astype(i32)
        colb = iot & (H - 1)

        def fire_in(kk, s):
            c1 = pltpu.async_copy(ft_h.at[sidx_all.at[kk]], ftb[s], sg[s])
            c2 = pltpu.async_copy(rd_h.at[didx_all.at[kk]], rdb[s], sg[s])
            return (c1, c2)

        def sync_out(kk, s):
            e0 = wid * EC + kk * BC
            pltpu.sync_copy(eeb[s % 2], a_h.at[pl.ds(e0, BC)])
            pltpu.sync_copy(ftb[s], rst_sp.at[didx_all.at[kk]], add=True)

        def compute(kk, s):
            e0 = wid * EC + kk * BC
            pltpu.sync_copy(ee_h.at[pl.ds(e0, BC)], eeb[s % 2])

            @plsc.parallel_loop(0, PAIRS_C, unroll=2)
            def pair(p):
                rows = jnp.full((L,), 2 * p, i32) + rowb
                eev = plsc.load_gather(eeb[s % 2], [rows, colb])
                rdv = plsc.load_gather(rdb[s], [rows, colb])
                av = eev * rdv
                plsc.store_scatter(eeb[s % 2], [rows, colb], av)
                j0 = 2 * p
                j1 = j0 + 1
                for jj in range(H):
                    b = av[jnp.full((L,), jj, i32)]
                    ftb[s][j0, pl.ds(jj * F, F)] = ftb[s][j0, pl.ds(jj * F, F)] * b
                for jj in range(H):
                    b = av[jnp.full((L,), H + jj, i32)]
                    ftb[s][j1, pl.ds(jj * F, F)] = ftb[s][j1, pl.ds(jj * F, F)] * b

        def step(q, c):
            k0 = 4 * q
            cps = [fire_in(k0 + s, s) for s in range(4)]
            for s in range(4):
                for cp in cps[s]:
                    cp.wait()
                compute(k0 + s, s)
                sync_out(k0 + s, s)
            return c

        lax.fori_loop(0, NCC // 4, step, 0)
        plsc.subcore_barrier()
        pltpu.sync_copy(rst_sp.at[pl.ds(sid * ROWS_R, ROWS_R)],
                        rst_h.at[cid, pl.ds(sid * ROWS_R, ROWS_R)])

    return k(src3, dst3, ee, rd, ft)


def _tc_add(p0, p1):
    def body(a_ref, b_ref, o_ref):
        o_ref[...] = a_ref[...] + b_ref[...]

    R = 1000
    return pl.pallas_call(
        body,
        grid=(N // R,),
        in_specs=[pl.BlockSpec((R, HF), lambda i: (i, 0))] * 2,
        out_specs=pl.BlockSpec((R, HF), lambda i: (i, 0)),
        out_shape=jax.ShapeDtypeStruct((N, HF), f32),
    )(p0, p1)


def kernel(feat, edge_index, W, attn_l, attn_r):
    src = edge_index[0]
    dst = edge_index[1]
    al = attn_l.reshape(H, F)
    ar = attn_r.reshape(H, F)
    eye = jnp.eye(H, dtype=f32)
    Al = (al[:, :, None] * eye[:, None, :]).reshape(HF, H)
    Ar = (ar[:, :, None] * eye[:, None, :]).reshape(HF, H)
    Alr = jnp.concatenate([Al, Ar], axis=1)

    ft, elr = _tc_prep(feat, W, Alr)
    ee, den = _sc_edge1(src.reshape(NW, NCA, BA), dst.reshape(NW, NCA, BA),
                        elr)
    rd = _sc_rdenom(den.reshape(NC, NPAD * H)).reshape(NPAD, H)
    a, rstp = _sc_edge2(src.reshape(NW, NCC, BC), dst.reshape(NW, NCC, BC),
                        ee, rd, ft)
    rst = _tc_add(rstp[0], rstp[1])
    return (rst.reshape(N, H, F), a)


# speedup vs baseline: 1.2798x; 1.2798x over previous
import functools

import jax
import jax.numpy as jnp
from jax import lax
from jax.experimental import pallas as pl
from jax.experimental.pallas import tpu as pltpu
from jax.experimental.pallas import tpu_sc as plsc

N = 10000
E = 320000
H = 8
F = 16
HF = H * F
NEG = 0.2
NC = 2
NS = 16
NW = NC * NS
L = 16
EC = E // NW
BA = 100
NCA = EC // BA
PAIRS_A = BA // 2
BC = 100
NCC = EC // BC
PAIRS_C = BC // 2
NPAD = 10240
ROWS_D = NPAD // NS
ROWS_R = N // NS
DW = NPAD * H // NW

f32 = jnp.float32
i32 = jnp.int32

_mesh = plsc.VectorSubcoreMesh(core_axis_name="c", subcore_axis_name="s")
_CP = pltpu.CompilerParams(needs_layout_passes=False, use_tc_tiling_on_sc=False)


def _tc_prep(feat, W, Alr):
    def body(f_ref, w_ref, alr_ref, ft_ref, elr_ref):
        ftb = jnp.dot(f_ref[...], w_ref[...], preferred_element_type=f32)
        ft_ref[...] = ftb
        elr_ref[...] = jnp.dot(ftb, alr_ref[...], preferred_element_type=f32)

    R = 1000
    return pl.pallas_call(
        body,
        grid=(N // R,),
        in_specs=[
            pl.BlockSpec((R, HF), lambda i: (i, 0)),
            pl.BlockSpec((HF, HF), lambda i: (0, 0)),
            pl.BlockSpec((HF, 2 * H), lambda i: (0, 0)),
        ],
        out_specs=[
            pl.BlockSpec((R, HF), lambda i: (i, 0)),
            pl.BlockSpec((R, 2 * H), lambda i: (i, 0)),
        ],
        out_shape=[
            jax.ShapeDtypeStruct((N, HF), f32),
            jax.ShapeDtypeStruct((N, 2 * H), f32),
        ],
    )(feat, W, Alr)


def _sc_edge1(src3, dst3, elr):
    @functools.partial(
        pl.kernel,
        out_type=(
            jax.ShapeDtypeStruct((E, H), f32),
            jax.ShapeDtypeStruct((NC, NPAD, H), f32),
        ),
        mesh=_mesh,
        scratch_types=[
            pltpu.VMEM((NCA, BA), i32),
            pltpu.VMEM((NCA, BA), i32),
            pltpu.VMEM((BA, 2 * H), f32),
            pltpu.VMEM((BA, 2 * H), f32),
            pltpu.VMEM((BA, 2 * H), f32),
            pltpu.VMEM((BA, 2 * H), f32),
            pltpu.VMEM((BA, H), f32),
            pltpu.VMEM((BA, H), f32),
            pltpu.VMEM_SHARED((NPAD, H), f32),
            pltpu.SemaphoreType.DMA,
            pltpu.SemaphoreType.DMA,
            pltpu.SemaphoreType.DMA,
            pltpu.SemaphoreType.DMA,
        ],
        compiler_params=_CP,
    )
    def k(src3_h, dst3_h, elr_h, ee_h, den_h,
          sidx_all, didx_all, srows0, srows1, drows0, drows1, ee0, ee1,
          den_sp, sg0, sg1, sw0, sw1):
        cid = lax.axis_index("c")
        sid = lax.axis_index("s")
        wid = cid * NS + sid
        srows = [srows0, srows1]
        drows = [drows0, drows1]
        eeb = [ee0, ee1]
        sg = [sg0, sg1]
        sw = [sw0, sw1]
        iot = lax.iota(i32, L)
        rowb = jnp.where(iot >= H, 1, 0).astype(i32)
        colb = iot & (H - 1)
        zed = jnp.zeros((L,), f32)

        @plsc.parallel_loop(0, PAIRS_A, unroll=1)
        def zloop(p):
            rows = jnp.full((L,), 2 * p, i32) + rowb
            plsc.store_scatter(ee0, [rows, colb], zed)

        for t in range(ROWS_D // BA):
            pltpu.sync_copy(ee0, den_sp.at[pl.ds(sid * ROWS_D + t * BA, BA)])
        pltpu.sync_copy(ee0.at[pl.ds(0, ROWS_D % BA)],
                        den_sp.at[pl.ds(sid * ROWS_D + (ROWS_D // BA) * BA,
                                        ROWS_D % BA)])
        pltpu.sync_copy(src3_h.at[wid], sidx_all)
        pltpu.sync_copy(dst3_h.at[wid], didx_all)
        plsc.subcore_barrier()

        def fire_in(kk, s):
            c1 = pltpu.async_copy(elr_h.at[sidx_all.at[kk]], srows[s], sg[s])
            c2 = pltpu.async_copy(elr_h.at[didx_all.at[kk]], drows[s], sg[s])
            return (c1, c2)

        def sync_out(kk, s):
            e0 = wid * EC + kk * BA
            pltpu.sync_copy(eeb[s], ee_h.at[pl.ds(e0, BA)])
            pltpu.sync_copy(eeb[s], den_sp.at[didx_all.at[kk]], add=True)

        def compute(s):
            @plsc.parallel_loop(0, PAIRS_A, unroll=2)
            def pair(p):
                rows = jnp.full((L,), 2 * p, i32) + rowb
                elv = plsc.load_gather(srows[s], [rows, colb])
                erv = plsc.load_gather(drows[s], [rows, colb + H])
                e = elv + erv
                e = jnp.where(e > 0, e, NEG * e)
                plsc.store_scatter(eeb[s], [rows, colb], jnp.exp(e))

        def step(half, c):
            k0 = 2 * half
            k1 = k0 + 1
            cps0 = fire_in(k0, 0)
            cps1 = fire_in(k1, 1)
            for cp in cps0:
                cp.wait()
            compute(0)
            sync_out(k0, 0)
            for cp in cps1:
                cp.wait()
            compute(1)
            sync_out(k1, 1)
            return c

        lax.fori_loop(0, NCA // 2, step, 0)
        plsc.subcore_barrier()
        pltpu.sync_copy(den_sp.at[pl.ds(sid * ROWS_D, ROWS_D)],
                        den_h.at[cid, pl.ds(sid * ROWS_D, ROWS_D)])

    return k(src3, dst3, elr)


def _sc_rdenom(dflat):
    @functools.partial(
        pl.kernel,
        out_type=jax.ShapeDtypeStruct((NPAD * H,), f32),
        mesh=_mesh,
        scratch_types=[
            pltpu.VMEM((DW,), f32),
            pltpu.VMEM((DW,), f32),
            pltpu.VMEM((DW,), f32),
        ],
        compiler_params=_CP,
    )
    def k(d_h, rd_h, b0, b1, ob):
        cid = lax.axis_index("c")
        sid = lax.axis_index("s")
        off = (cid * NS + sid) * DW
        pltpu.sync_copy(d_h.at[0, pl.ds(off, DW)], b0)
        pltpu.sync_copy(d_h.at[1, pl.ds(off, DW)], b1)

        def it(j, c):
            v = b0[pl.ds(j * L, L)] + b1[pl.ds(j * L, L)]
            ob[pl.ds(j * L, L)] = 1.0 / v
            return c

        lax.fori_loop(0, DW // L, it, 0)
        pltpu.sync_copy(ob, rd_h.at[pl.ds(off, DW)])

    return k(dflat)


def _sc_edge2(src3, dst3, ee, rd, ft):
    @functools.partial(
        pl.kernel,
        out_type=(
            jax.ShapeDtypeStruct((E, H), f32),
            jax.ShapeDtypeStruct((NC, N, HF), f32),
        ),
        mesh=_mesh,
        scratch_types=[
            pltpu.VMEM((NCC, BC), i32),
            pltpu.VMEM((NCC, BC), i32),
            pltpu.VMEM((BC, H), f32),
            pltpu.VMEM((BC, H), f32),
            pltpu.VMEM((BC, H), f32),
            pltpu.VMEM((BC, H), f32),
            pltpu.VMEM((BC, HF), f32),
            pltpu.VMEM((BC, HF), f32),
            pltpu.VMEM_SHARED((N, HF), f32),
            pltpu.SemaphoreType.DMA,
            pltpu.SemaphoreType.DMA,
        ],
        compiler_params=_CP,
    )
    def k(src3_h, dst3_h, ee_h, rd_h, ft_h, a_h, rst_h,
          sidx_all, didx_all, ee0, ee1, rd0, rd1,
          ft0, ft1, rst_sp, sg0, sg1):
        cid = lax.axis_index("c")
        sid = lax.axis_index("s")
        wid = cid * NS + sid
        zed = jnp.zeros((L,), f32)

        @plsc.parallel_loop(0, BC, unroll=1)
        def zloop(j):
            for jj in range(H):
                ft0[j, pl.ds(jj * F, F)] = zed

        for t in range(ROWS_R // BC):
            pltpu.sync_copy(ft0, rst_sp.at[pl.ds(sid * ROWS_R + t * BC, BC)])
        pltpu.sync_copy(ft0.at[pl.ds(0, ROWS_R % BC)],
                        rst_sp.at[pl.ds(sid * ROWS_R + (ROWS_R // BC) * BC,
                                        ROWS_R % BC)])
        pltpu.sync_copy(src3_h.at[wid], sidx_all)
        pltpu.sync_copy(dst3_h.at[wid], didx_all)
        plsc.subcore_barrier()
        eeb = [ee0, ee1]
        rdb = [rd0, rd1]
        ftb = [ft0, ft1]
        sg = [sg0, sg1]
        iot = lax.iota(i32, L)
        rowb = jnp.where(iot >= H, 1, 0).astype(i32)
        colb = iot & (H - 1)

        def fire_in(kk, s):
            e0 = wid * EC + kk * BC
            c1 = pltpu.async_copy(ft_h.at[sidx_all.at[kk]], ftb[s], sg[s])
            c2 = pltpu.async_copy(rd_h.at[didx_all.at[kk]], rdb[s], sg[s])
            c3 = pltpu.async_copy(ee_h.at[pl.ds(e0, BC)], eeb[s], sg[s])
            return (c1, c2, c3)

        def sync_out(kk, s):
            e0 = wid * EC + kk * BC
            pltpu.sync_copy(eeb[s], a_h.at[pl.ds(e0, BC)])
            pltpu.sync_copy(ftb[s], rst_sp.at[didx_all.at[kk]], add=True)

        def compute(s):
            @plsc.parallel_loop(0, PAIRS_C, unroll=1)
            def pair(p):
                rows = jnp.full((L,), 2 * p, i32) + rowb
                eev = plsc.load_gather(eeb[s], [rows, colb])
                rdv = plsc.load_gather(rdb[s], [rows, colb])
                av = eev * rdv
                plsc.store_scatter(eeb[s], [rows, colb], av)
                j0 = 2 * p
                j1 = j0 + 1
                for jj in range(H):
                    b = av[jnp.full((L,), jj, i32)]
                    ftb[s][j0, pl.ds(jj * F, F)] = ftb[s][j0, pl.ds(jj * F, F)] * b
                for jj in range(H):
                    b = av[jnp.full((L,), H + jj, i32)]
                    ftb[s][j1, pl.ds(jj * F, F)] = ftb[s][j1, pl.ds(jj * F, F)] * b

        def step(half, c):
            k0 = 2 * half
            k1 = k0 + 1
            cps0 = fire_in(k0, 0)
            cps1 = fire_in(k1, 1)
            for cp in cps0:
                cp.wait()
            compute(0)
            sync_out(k0, 0)
            for cp in cps1:
                cp.wait()
            compute(1)
            sync_out(k1, 1)
            return c

        lax.fori_loop(0, NCC // 2, step, 0)
        plsc.subcore_barrier()
        pltpu.sync_copy(rst_sp.at[pl.ds(sid * ROWS_R, ROWS_R)],
                        rst_h.at[cid, pl.ds(sid * ROWS_R, ROWS_R)])

    return k(src3, dst3, ee, rd, ft)


def _tc_add(p0, p1):
    def body(a_ref, b_ref, o_ref):
        o_ref[...] = a_ref[...] + b_ref[...]

    R = 1000
    return pl.pallas_call(
        body,
        grid=(N // R,),
        in_specs=[pl.BlockSpec((R, HF), lambda i: (i, 0))] * 2,
        out_specs=pl.BlockSpec((R, HF), lambda i: (i, 0)),
        out_shape=jax.ShapeDtypeStruct((N, HF), f32),
    )(p0, p1)


def kernel(feat, edge_index, W, attn_l, attn_r):
    src = edge_index[0]
    dst = edge_index[1]
    al = attn_l.reshape(H, F)
    ar = attn_r.reshape(H, F)
    eye = jnp.eye(H, dtype=f32)
    Al = (al[:, :, None] * eye[:, None, :]).reshape(HF, H)
    Ar = (ar[:, :, None] * eye[:, None, :]).reshape(HF, H)
    Alr = jnp.concatenate([Al, Ar], axis=1)

    ft, elr = _tc_prep(feat, W, Alr)
    ee, den = _sc_edge1(src.reshape(NW, NCA, BA), dst.reshape(NW, NCA, BA),
                        elr)
    rd = _sc_rdenom(den.reshape(NC, NPAD * H)).reshape(NPAD, H)
    a, rstp = _sc_edge2(src.reshape(NW, NCC, BC), dst.reshape(NW, NCC, BC),
                        ee, rd, ft)
    rst = _tc_add(rstp[0], rstp[1])
    return (rst.reshape(N, H, F), a)
